# baseline (speedup 1.0000x reference)
import functools

import jax
import jax.numpy as jnp
from jax import lax
from jax.experimental import pallas as pl
from jax.experimental.pallas import tpu as pltpu
from jax.experimental.pallas import tpu_sc as plsc

NUM_USERS = 100000
NUM_ITEMS = 100000
EMB_IN = 128
EMB_OUT = 300
BATCH = 16384

_NPIPE = 2
_BCHUNK = BATCH // _NPIPE

_NC = 2
_NS = 16
_NW = _NC * _NS
_BPW = _BCHUNK // _NW
_CHUNK = 128
_NSTREAM = _BPW // _CHUNK

_mesh = plsc.VectorSubcoreMesh(core_axis_name="c", subcore_axis_name="s")
_NUNIT = 2 * _NSTREAM


def _make_sc_gather(chunk):
    base0 = chunk * _BCHUNK

    @functools.partial(
        pl.kernel,
        mesh=_mesh,
        out_type=[
            jax.ShapeDtypeStruct((_BCHUNK, EMB_IN), jnp.float32),
            jax.ShapeDtypeStruct((_BCHUNK, EMB_IN), jnp.float32),
        ],
        scratch_types=[
            pltpu.VMEM((_BPW,), jnp.int32),
            pltpu.VMEM((_BPW,), jnp.int32),
            pltpu.VMEM((_BPW, EMB_IN), jnp.float32),
            pltpu.VMEM((_BPW, EMB_IN), jnp.float32),
        ] + [pltpu.SemaphoreType.DMA] * _NUNIT,
    )
    def k(ut_hbm, it_hbm, uidx_hbm, iidx_hbm, ue_out, ie_out,
          uidx_v, iidx_v, urows_v, irows_v, *sems):
        wid = lax.axis_index("s") * _NC + lax.axis_index("c")
        gbase = base0 + wid * _BPW
        obase = wid * _BPW
        pltpu.sync_copy(uidx_hbm.at[pl.ds(gbase, _BPW)], uidx_v)
        pltpu.sync_copy(iidx_hbm.at[pl.ds(gbase, _BPW)], iidx_v)
        units = []
        for j in range(_NSTREAM):
            units.append((ut_hbm, uidx_v, urows_v, ue_out, j, sems[j]))
        for j in range(_NSTREAM):
            units.append((it_hbm, iidx_v, irows_v, ie_out, j,
                          sems[_NSTREAM + j]))
        gathers = [
            pltpu.async_copy(tbl.at[idxv.at[pl.ds(j * _CHUNK, _CHUNK)]],
                             rows.at[pl.ds(j * _CHUNK, _CHUNK)], sem)
            for tbl, idxv, rows, out, j, sem in units
        ]
        scatters = []
        for (tbl, idxv, rows, out, j, sem), g in zip(units, gathers):
            g.wait()
            scatters.append(
                pltpu.async_copy(rows.at[pl.ds(j * _CHUNK, _CHUNK)],
                                 out.at[pl.ds(obase + j * _CHUNK, _CHUNK)],
                                 sem))
        for s in scatters:
            s.wait()

    return k


_BLK = 1024
_NPAD = 384
_EPS = 1e-8


_NT = (((1,), (1,)), ((), ()))


def _tc_body(ue_ref, ie_ref, wt_ref, bt_ref, out_ref):
    ut = jax.lax.dot_general(wt_ref[...], ue_ref[...], _NT,
                             preferred_element_type=jnp.float32)
    vt = jax.lax.dot_general(wt_ref[...], ie_ref[...], _NT,
                             preferred_element_type=jnp.float32)
    bt = bt_ref[...]
    ut = jnp.maximum(ut + bt, 0.0)
    vt = jnp.maximum(vt + bt, 0.0)
    dot = jnp.sum(ut * vt, axis=0)
    nu2 = jnp.sum(ut * ut, axis=0)
    nv2 = jnp.sum(vt * vt, axis=0)
    denom2 = jnp.maximum(nu2, _EPS * _EPS) * jnp.maximum(nv2, _EPS * _EPS)
    out_ref[0, 0, :] = dot * jax.lax.rsqrt(denom2)


_tc_call = pl.pallas_call(
    _tc_body,
    grid=(_BCHUNK // _BLK,),
    in_specs=[
        pl.BlockSpec((_BLK, EMB_IN), lambda i: (i, 0)),
        pl.BlockSpec((_BLK, EMB_IN), lambda i: (i, 0)),
        pl.BlockSpec((_NPAD, EMB_IN), lambda i: (0, 0)),
        pl.BlockSpec((_NPAD, 1), lambda i: (0, 0)),
    ],
    out_specs=pl.BlockSpec((1, 1, _BLK), lambda i: (i, 0, 0)),
    out_shape=jax.ShapeDtypeStruct((_BCHUNK // _BLK, 1, _BLK), jnp.float32),
)


_sc_gathers = [_make_sc_gather(c) for c in range(_NPIPE)]


def kernel(user_indices, item_indices, user_table, item_table, W, b):
    uidx = user_indices.astype(jnp.int32)
    iidx = item_indices.astype(jnp.int32)
    w_pad = jnp.pad(W, ((0, 0), (0, _NPAD - EMB_OUT))).T
    b_pad = jnp.pad(b, (0, _NPAD - EMB_OUT)).reshape(_NPAD, 1)
    scores = []
    for c in range(_NPIPE):
        ue, ie = _sc_gathers[c](user_table, item_table, uidx, iidx)
        scores.append(_tc_call(ue, ie, w_pad, b_pad).reshape(_BCHUNK))
    return jnp.concatenate(scores)


# speedup vs baseline: 1.0284x; 1.0284x over previous
import functools

import jax
import jax.numpy as jnp
from jax import lax
from jax.experimental import pallas as pl
from jax.experimental.pallas import tpu as pltpu
from jax.experimental.pallas import tpu_sc as plsc

NUM_USERS = 100000
NUM_ITEMS = 100000
EMB_IN = 128
EMB_OUT = 300
BATCH = 16384

_NPIPE = 2
_BCHUNK = BATCH // _NPIPE

_NC = 2
_NS = 16
_NW = _NC * _NS
_BPW = _BCHUNK // _NW
_CHUNK = 128
_NSTREAM = _BPW // _CHUNK

_mesh = plsc.VectorSubcoreMesh(core_axis_name="c", subcore_axis_name="s")
_NUNIT = 2 * _NSTREAM


def _make_sc_gather(chunk):
    base0 = chunk * _BCHUNK

    @functools.partial(
        pl.kernel,
        mesh=_mesh,
        out_type=[
            jax.ShapeDtypeStruct((_BCHUNK, EMB_IN), jnp.float32),
            jax.ShapeDtypeStruct((_BCHUNK, EMB_IN), jnp.float32),
        ],
        scratch_types=[
            pltpu.VMEM((_BPW,), jnp.int32),
            pltpu.VMEM((_BPW,), jnp.int32),
            pltpu.VMEM((_BPW, EMB_IN), jnp.float32),
            pltpu.VMEM((_BPW, EMB_IN), jnp.float32),
        ] + [pltpu.SemaphoreType.DMA] * _NUNIT,
    )
    def k(ut_hbm, it_hbm, uidx_hbm, iidx_hbm, ue_out, ie_out,
          uidx_v, iidx_v, urows_v, irows_v, *sems):
        wid = lax.axis_index("s") * _NC + lax.axis_index("c")
        gbase = base0 + wid * _BPW
        obase = wid * _BPW
        pltpu.sync_copy(uidx_hbm.at[pl.ds(gbase, _BPW)], uidx_v)
        pltpu.sync_copy(iidx_hbm.at[pl.ds(gbase, _BPW)], iidx_v)
        units = []
        for j in range(_NSTREAM):
            units.append((ut_hbm, uidx_v, urows_v, ue_out, j, sems[j]))
        for j in range(_NSTREAM):
            units.append((it_hbm, iidx_v, irows_v, ie_out, j,
                          sems[_NSTREAM + j]))
        gathers = [
            pltpu.async_copy(tbl.at[idxv.at[pl.ds(j * _CHUNK, _CHUNK)]],
                             rows.at[pl.ds(j * _CHUNK, _CHUNK)], sem)
            for tbl, idxv, rows, out, j, sem in units
        ]
        scatters = []
        for (tbl, idxv, rows, out, j, sem), g in zip(units, gathers):
            g.wait()
            scatters.append(
                pltpu.async_copy(rows.at[pl.ds(j * _CHUNK, _CHUNK)],
                                 out.at[pl.ds(obase + j * _CHUNK, _CHUNK)],
                                 sem))
        for s in scatters:
            s.wait()

    return k


_BLK = 2048
_NPAD = 384
_EPS = 1e-8


_NT = (((1,), (1,)), ((), ()))


def _tc_body(ue_ref, ie_ref, wt_ref, bt_ref, out_ref):
    ut = jax.lax.dot_general(wt_ref[...], ue_ref[...], _NT,
                             preferred_element_type=jnp.float32)
    vt = jax.lax.dot_general(wt_ref[...], ie_ref[...], _NT,
                             preferred_element_type=jnp.float32)
    bt = bt_ref[...]
    ut = jnp.maximum(ut + bt, 0.0)
    vt = jnp.maximum(vt + bt, 0.0)
    dot = jnp.sum(ut * vt, axis=0)
    nu2 = jnp.sum(ut * ut, axis=0)
    nv2 = jnp.sum(vt * vt, axis=0)
    denom2 = jnp.maximum(nu2, _EPS * _EPS) * jnp.maximum(nv2, _EPS * _EPS)
    out_ref[0, 0, :] = dot * jax.lax.rsqrt(denom2)


_tc_call = pl.pallas_call(
    _tc_body,
    grid=(_BCHUNK // _BLK,),
    in_specs=[
        pl.BlockSpec((_BLK, EMB_IN), lambda i: (i, 0)),
        pl.BlockSpec((_BLK, EMB_IN), lambda i: (i, 0)),
        pl.BlockSpec((_NPAD, EMB_IN), lambda i: (0, 0)),
        pl.BlockSpec((_NPAD, 1), lambda i: (0, 0)),
    ],
    out_specs=pl.BlockSpec((1, 1, _BLK), lambda i: (i, 0, 0)),
    out_shape=jax.ShapeDtypeStruct((_BCHUNK // _BLK, 1, _BLK), jnp.float32),
)


_sc_gathers = [_make_sc_gather(c) for c in range(_NPIPE)]


def kernel(user_indices, item_indices, user_table, item_table, W, b):
    uidx = user_indices.astype(jnp.int32)
    iidx = item_indices.astype(jnp.int32)
    w_pad = jnp.pad(W, ((0, 0), (0, _NPAD - EMB_OUT))).T
    b_pad = jnp.pad(b, (0, _NPAD - EMB_OUT)).reshape(_NPAD, 1)
    scores = []
    for c in range(_NPIPE):
        ue, ie = _sc_gathers[c](user_table, item_table, uidx, iidx)
        scores.append(_tc_call(ue, ie, w_pad, b_pad).reshape(_BCHUNK))
    return jnp.concatenate(scores)


# speedup vs baseline: 1.0733x; 1.0437x over previous
import functools

import jax
import jax.numpy as jnp
from jax import lax
from jax.experimental import pallas as pl
from jax.experimental.pallas import tpu as pltpu
from jax.experimental.pallas import tpu_sc as plsc

NUM_USERS = 100000
NUM_ITEMS = 100000
EMB_IN = 128
EMB_OUT = 300
BATCH = 16384

_NC = 2
_NS = 16
_NW = _NC * _NS
_BPW = BATCH // _NW
_CHUNK = 128
_NSTREAM = _BPW // _CHUNK
_NUNIT = 2 * _NSTREAM
_RING = 6

_mesh = plsc.VectorSubcoreMesh(core_axis_name="c", subcore_axis_name="s")


@functools.partial(
    pl.kernel,
    mesh=_mesh,
    out_type=[
        jax.ShapeDtypeStruct((BATCH, EMB_IN), jnp.float32),
        jax.ShapeDtypeStruct((BATCH, EMB_IN), jnp.float32),
    ],
    scratch_types=[
        pltpu.VMEM((_BPW,), jnp.int32),
        pltpu.VMEM((_BPW,), jnp.int32),
    ] + [pltpu.VMEM((_CHUNK, EMB_IN), jnp.float32)] * _RING
      + [pltpu.SemaphoreType.DMA] * (2 * _RING),
)
def _sc_gather(ut_hbm, it_hbm, uidx_hbm, iidx_hbm, ue_out, ie_out,
               uidx_v, iidx_v, *rest):
    bufs = rest[:_RING]
    gsems = rest[_RING:2 * _RING]
    ssems = rest[2 * _RING:]
    wid = lax.axis_index("s") * _NC + lax.axis_index("c")
    gbase = wid * _BPW
    pltpu.sync_copy(uidx_hbm.at[pl.ds(gbase, _BPW)], uidx_v)
    pltpu.sync_copy(iidx_hbm.at[pl.ds(gbase, _BPW)], iidx_v)
    units = ([(ut_hbm, uidx_v, ue_out, j) for j in range(_NSTREAM)]
             + [(it_hbm, iidx_v, ie_out, j) for j in range(_NSTREAM)])
    g = [None] * _NUNIT
    s = [None] * _NUNIT
    for k, (tbl, idxv, out, j) in enumerate(units):
        r = k % _RING
        if k >= _RING:
            ptbl, pidxv, pout, pj = units[k - _RING]
            g[k - _RING].wait()
            s[k - _RING] = pltpu.async_copy(
                bufs[r], pout.at[pl.ds(gbase + pj * _CHUNK, _CHUNK)],
                ssems[r])
            s[k - _RING].wait()
        g[k] = pltpu.async_copy(
            tbl.at[idxv.at[pl.ds(j * _CHUNK, _CHUNK)]], bufs[r], gsems[r])
    for k in range(_NUNIT - _RING, _NUNIT):
        r = k % _RING
        tbl, idxv, out, j = units[k]
        g[k].wait()
        s[k] = pltpu.async_copy(
            bufs[r], out.at[pl.ds(gbase + j * _CHUNK, _CHUNK)], ssems[r])
    for k in range(_NUNIT - _RING, _NUNIT):
        s[k].wait()


_BLK = 2048
_NPAD = 384
_EPS = 1e-8
_NT = (((1,), (1,)), ((), ()))


def _tc_body(ue_ref, ie_ref, wt_ref, bt_ref, out_ref):
    ut = jax.lax.dot_general(wt_ref[...], ue_ref[...], _NT,
                             preferred_element_type=jnp.float32)
    vt = jax.lax.dot_general(wt_ref[...], ie_ref[...], _NT,
                             preferred_element_type=jnp.float32)
    bt = bt_ref[...]
    ut = jnp.maximum(ut + bt, 0.0)
    vt = jnp.maximum(vt + bt, 0.0)
    dot = jnp.sum(ut * vt, axis=0)
    nu2 = jnp.sum(ut * ut, axis=0)
    nv2 = jnp.sum(vt * vt, axis=0)
    denom2 = jnp.maximum(nu2, _EPS * _EPS) * jnp.maximum(nv2, _EPS * _EPS)
    out_ref[0, 0, :] = dot * jax.lax.rsqrt(denom2)


_tc_call = pl.pallas_call(
    _tc_body,
    grid=(BATCH // _BLK,),
    in_specs=[
        pl.BlockSpec((_BLK, EMB_IN), lambda i: (i, 0)),
        pl.BlockSpec((_BLK, EMB_IN), lambda i: (i, 0)),
        pl.BlockSpec((_NPAD, EMB_IN), lambda i: (0, 0)),
        pl.BlockSpec((_NPAD, 1), lambda i: (0, 0)),
    ],
    out_specs=pl.BlockSpec((1, 1, _BLK), lambda i: (i, 0, 0)),
    out_shape=jax.ShapeDtypeStruct((BATCH // _BLK, 1, _BLK), jnp.float32),
)


def kernel(user_indices, item_indices, user_table, item_table, W, b):
    uidx = user_indices.astype(jnp.int32)
    iidx = item_indices.astype(jnp.int32)
    w_pad = jnp.pad(W, ((0, 0), (0, _NPAD - EMB_OUT))).T
    b_pad = jnp.pad(b, (0, _NPAD - EMB_OUT)).reshape(_NPAD, 1)
    ue, ie = _sc_gather(user_table, item_table, uidx, iidx)
    return _tc_call(ue, ie, w_pad, b_pad).reshape(BATCH)
